# Initial kernel scaffold; baseline (speedup 1.0000x reference)
import jax
import jax.numpy as jnp
from jax.experimental import pallas as pl


def kernel(indices, embed, W, b):
    raise NotImplementedError("write your pallas kernel here")


# speedup vs baseline: 39.4353x; 39.4353x over previous
import functools

import jax
import jax.numpy as jnp
import numpy as np
from jax import lax
from jax.experimental import pallas as pl
from jax.experimental.pallas import tpu as pltpu
from jax.experimental.pallas import tpu_sc as plsc

_NC = 2
_NS = 16
_NW = _NC * _NS
_LANES = 16


def _tc_body(e_ref, w_ref, b_ref, v_ref, lse_ref, m_s, s_s):
    j = pl.program_id(0)
    v = jnp.dot(e_ref[...], w_ref[...], preferred_element_type=jnp.float32)
    v = v + b_ref[...]
    v_ref[...] = v
    bm = jnp.max(v)
    prev_m = jnp.where(j == 0, -jnp.inf, m_s[0])
    prev_s = jnp.where(j == 0, 0.0, s_s[0])
    new_m = jnp.maximum(prev_m, bm)
    new_s = prev_s * jnp.exp(prev_m - new_m) + jnp.sum(jnp.exp(v - new_m))
    m_s[0] = new_m
    s_s[0] = new_s

    @pl.when(j == pl.num_programs(0) - 1)
    def _():
        lse_ref[0, 0] = new_m + jnp.log(new_s)


def _tc_logits_lse(e, W, b2):
    D, K = W.shape
    blk = 1024
    grid = (K // blk,)
    return pl.pallas_call(
        _tc_body,
        grid=grid,
        in_specs=[
            pl.BlockSpec((1, D), lambda j: (0, 0)),
            pl.BlockSpec((D, blk), lambda j: (0, j)),
            pl.BlockSpec((1, blk), lambda j: (0, j)),
        ],
        out_specs=[
            pl.BlockSpec((1, blk), lambda j: (0, j)),
            pl.BlockSpec((1, 1), lambda j: (0, 0), memory_space=pltpu.SMEM),
        ],
        out_shape=[
            jax.ShapeDtypeStruct((1, K), jnp.float32),
            jax.ShapeDtypeStruct((1, 1), jnp.float32),
        ],
        scratch_shapes=[
            pltpu.SMEM((1,), jnp.float32),
            pltpu.SMEM((1,), jnp.float32),
        ],
    )(e, W, b2)


def _sc_body(idx_hbm, mask_hbm, v_hbm, out_hbm, idx_v, m_v, v_v, o_v):
    c = lax.axis_index("c")
    s = lax.axis_index("s")
    wid = s * _NC + c
    chunk = 512
    base = wid * chunk
    base_m = (wid % 2) * chunk
    pltpu.sync_copy(idx_hbm.at[pl.ds(base, chunk)], idx_v)
    pltpu.sync_copy(mask_hbm.at[pl.ds(base_m, chunk)], m_v)
    pltpu.sync_copy(v_hbm, v_v)

    def body(i, acc):
        ii = i * _LANES
        idx = idx_v[pl.ds(ii, _LANES)]
        mm = m_v[pl.ds(ii, _LANES)]
        vals = plsc.load_gather(v_v, [idx])
        return acc + mm * vals

    acc = lax.fori_loop(0, chunk // _LANES, body, jnp.zeros((_LANES,), jnp.float32))
    o_v[0, :] = acc
    pltpu.sync_copy(o_v, out_hbm.at[pl.ds(wid, 1)])


def _sc_masked_gather_sum(flat_idx, mask_cols, v_flat):
    K = v_flat.shape[0]
    mesh = plsc.VectorSubcoreMesh(core_axis_name="c", subcore_axis_name="s")
    fn = functools.partial(
        pl.kernel,
        mesh=mesh,
        compiler_params=pltpu.CompilerParams(needs_layout_passes=False),
        out_type=jax.ShapeDtypeStruct((_NW, _LANES), jnp.float32),
        scratch_types=[
            pltpu.VMEM((512,), jnp.int32),
            pltpu.VMEM((512,), jnp.float32),
            pltpu.VMEM((K,), jnp.float32),
            pltpu.VMEM((1, _LANES), jnp.float32),
        ],
    )(_sc_body)
    return fn(flat_idx, mask_cols, v_flat)


def kernel(indices, embed, W, b):
    Bb, Hh, Ww = indices.shape
    L = Hh * Ww
    D, K = W.shape
    num_masked = int(np.cos(0.5 * np.pi / 2) * L)
    perm = jax.random.permutation(jax.random.key(42), L)
    mask_idx = perm[:num_masked]
    mask_cols = jnp.zeros((L,), jnp.float32).at[mask_idx].set(1.0)

    e = lax.slice(embed, (K, 0), (K + 1, D))
    b2 = b.reshape(1, K)
    v, lse = _tc_logits_lse(e, W, b2)

    flat_idx = indices.reshape(-1)
    partials = _sc_masked_gather_sum(flat_idx, mask_cols, v.reshape(K))
    total = jnp.sum(partials)
    return lse[0, 0] - total / (Bb * num_masked)
